# Initial kernel scaffold; baseline (speedup 1.0000x reference)
import jax
import jax.numpy as jnp
from jax.experimental import pallas as pl


def kernel(length, rel_pos_embeddings):
    raise NotImplementedError("write your pallas kernel here")


# speedup vs baseline: 13.0122x; 13.0122x over previous
import functools

import jax
import jax.numpy as jnp
from jax import lax
from jax.experimental import pallas as pl
from jax.experimental.pallas import tpu as pltpu
from jax.experimental.pallas import tpu_sc as plsc

D_MODEL = 128
MAX_REL = 32
LENGTH = 1024
VOCAB = 2 * MAX_REL + 1
TROWS = 2048

_NUM_CORES = 2
_NUM_SUBCORES = 16
_NUM_WORKERS = _NUM_CORES * _NUM_SUBCORES
_ROWS_PER_WORKER = LENGTH // _NUM_WORKERS


def _template_body(tab_ref, out_ref):
    k = lax.broadcasted_iota(jnp.int32, (TROWS, 128), 0)
    v = lax.broadcasted_iota(jnp.int32, (TROWS, 128), 1)
    idx = jnp.clip(k - (LENGTH - 1), -MAX_REL, MAX_REL) + MAX_REL
    onehot = (idx == v).astype(jnp.float32)
    out_ref[...] = lax.dot_general(
        onehot, tab_ref[...],
        dimension_numbers=(((1,), (0,)), ((), ())),
        preferred_element_type=jnp.float32,
    )


def _build_template(tab_padded):
    return pl.pallas_call(
        _template_body,
        out_shape=jax.ShapeDtypeStruct((TROWS, D_MODEL), jnp.float32),
    )(tab_padded)


def _fanout_body(tmpl_hbm, out_hbm, tmpl_sh):
    c = lax.axis_index("c")
    s = lax.axis_index("s")

    @pl.when(s == 0)
    def _():
        pltpu.sync_copy(tmpl_hbm, tmpl_sh)

    plsc.subcore_barrier()

    wid = s * _NUM_CORES + c
    base = wid * _ROWS_PER_WORKER

    def row(r, carry):
        i = base + r
        start = (LENGTH - 1) - i
        pltpu.sync_copy(tmpl_sh.at[pl.ds(start, LENGTH)], out_hbm.at[i])
        return carry

    lax.fori_loop(0, _ROWS_PER_WORKER, row, 0)


@functools.cache
def _fanout():
    return pl.kernel(
        _fanout_body,
        out_type=jax.ShapeDtypeStruct((LENGTH, LENGTH, D_MODEL), jnp.float32),
        mesh=plsc.VectorSubcoreMesh(core_axis_name="c", subcore_axis_name="s",
                                    num_cores=_NUM_CORES,
                                    num_subcores=_NUM_SUBCORES),
        scratch_types=[pltpu.VMEM_SHARED((TROWS, D_MODEL), jnp.float32)],
    )


def kernel(length, rel_pos_embeddings):
    del length
    tab_padded = jnp.zeros((128, D_MODEL), jnp.float32)
    tab_padded = lax.dynamic_update_slice(
        tab_padded, rel_pos_embeddings.astype(jnp.float32), (0, 0))
    tmpl = _build_template(tab_padded)
    return _fanout()(tmpl)


# speedup vs baseline: 23.0716x; 1.7731x over previous
import functools

import jax
import jax.numpy as jnp
from jax import lax
from jax.experimental import pallas as pl
from jax.experimental.pallas import tpu as pltpu
from jax.experimental.pallas import tpu_sc as plsc

D_MODEL = 128
MAX_REL = 32
LENGTH = 1024
TROWS = 2048

_TC_BLOCK_ROWS = 8


def _template_body(tab_ref, out_ref):
    k = lax.broadcasted_iota(jnp.int32, (TROWS, 128), 0)
    v = lax.broadcasted_iota(jnp.int32, (TROWS, 128), 1)
    idx = jnp.clip(k - (LENGTH - 1), -MAX_REL, MAX_REL) + MAX_REL
    onehot = (idx == v).astype(jnp.float32)
    out_ref[...] = lax.dot_general(
        onehot, tab_ref[...],
        dimension_numbers=(((1,), (0,)), ((), ())),
        preferred_element_type=jnp.float32,
    )


def _build_template(tab_padded):
    return pl.pallas_call(
        _template_body,
        out_shape=jax.ShapeDtypeStruct((TROWS, D_MODEL), jnp.float32),
    )(tab_padded)


def _tc_fanout_body(tmpl_ref, out_ref):
    pid = pl.program_id(0)
    for k in range(_TC_BLOCK_ROWS):
        i = pid * _TC_BLOCK_ROWS + k
        start = (LENGTH - 1) - i
        pltpu.sync_copy(tmpl_ref.at[pl.ds(start, LENGTH), :], out_ref.at[k])


def _tc_fanout(tmpl):
    return pl.pallas_call(
        _tc_fanout_body,
        grid=(LENGTH // _TC_BLOCK_ROWS,),
        in_specs=[pl.BlockSpec((TROWS, D_MODEL), lambda i: (0, 0))],
        out_specs=pl.BlockSpec((_TC_BLOCK_ROWS, LENGTH, D_MODEL),
                               lambda i: (i, 0, 0)),
        out_shape=jax.ShapeDtypeStruct((LENGTH, LENGTH, D_MODEL), jnp.float32),
    )(tmpl)


def kernel(length, rel_pos_embeddings):
    del length
    tab_padded = jnp.zeros((128, D_MODEL), jnp.float32)
    tab_padded = lax.dynamic_update_slice(
        tab_padded, rel_pos_embeddings.astype(jnp.float32), (0, 0))
    tmpl = _build_template(tab_padded)
    return _tc_fanout(tmpl)
